# baseline (speedup 1.0000x reference)
import jax
import jax.numpy as jnp
from jax.experimental import pallas as pl
from jax.experimental.pallas import tpu as pltpu

N = 10000
F_IN = 128
F_HID = 64
BM = 400
NBUF = 3
NOUT = 2
NSTEPS = N // BM


def _gcn_kernel(x_ref, w_ref, b_ref, adj_ref, logp_ref, embed_ref,
                buf_ref, support_ref, stage_logp_ref, stage_embed_ref,
                sem_ref, out_sem_ref):
    def adj_copy(step, slot):
        return pltpu.make_async_copy(
            adj_ref.at[pl.ds(step * BM, BM), :],
            buf_ref.at[slot],
            sem_ref.at[slot],
        )

    def out_copies(step, oslot):
        return (
            pltpu.make_async_copy(
                stage_logp_ref.at[oslot],
                logp_ref.at[pl.ds(step * BM, BM), :],
                out_sem_ref.at[0, oslot],
            ),
            pltpu.make_async_copy(
                stage_embed_ref.at[oslot],
                embed_ref.at[pl.ds(step * BM, BM), :],
                out_sem_ref.at[1, oslot],
            ),
        )

    for s in range(NBUF):
        adj_copy(s, s).start()

    support_ref[:] = jnp.dot(
        x_ref[:], w_ref[:], preferred_element_type=jnp.float32
    )

    def body(i, _):
        slot = jax.lax.rem(i, NBUF)
        oslot = jax.lax.rem(i, NOUT)

        @pl.when(i >= NOUT)
        def _():
            cl, ce = out_copies(i - NOUT, oslot)
            cl.wait()
            ce.wait()

        adj_copy(i, slot).wait()
        out = jnp.dot(buf_ref[slot], support_ref[:],
                      preferred_element_type=jnp.float32)
        out = out + b_ref[:]
        stage_embed_ref[oslot] = out
        m = jnp.max(out, axis=1, keepdims=True)
        lse = jnp.log(jnp.sum(jnp.exp(out - m), axis=1, keepdims=True)) + m
        stage_logp_ref[oslot] = out - lse
        cl, ce = out_copies(i, oslot)
        cl.start()
        ce.start()

        @pl.when(i + NBUF < NSTEPS)
        def _():
            adj_copy(i + NBUF, slot).start()

        return 0

    jax.lax.fori_loop(0, NSTEPS, body, 0)

    for j in range(NOUT):
        step = NSTEPS - NOUT + j
        cl, ce = out_copies(step, step % NOUT)
        cl.wait()
        ce.wait()


def kernel(x, adj, W, b):
    b2 = b.reshape(1, F_HID)
    logp, embed = pl.pallas_call(
        _gcn_kernel,
        in_specs=[
            pl.BlockSpec(memory_space=pltpu.VMEM),
            pl.BlockSpec(memory_space=pltpu.VMEM),
            pl.BlockSpec(memory_space=pltpu.VMEM),
            pl.BlockSpec(memory_space=pl.ANY),
        ],
        out_specs=[
            pl.BlockSpec(memory_space=pl.ANY),
            pl.BlockSpec(memory_space=pl.ANY),
        ],
        out_shape=[
            jax.ShapeDtypeStruct((N, F_HID), jnp.float32),
            jax.ShapeDtypeStruct((N, F_HID), jnp.float32),
        ],
        scratch_shapes=[
            pltpu.VMEM((NBUF, BM, N), jnp.float32),
            pltpu.VMEM((N, F_HID), jnp.float32),
            pltpu.VMEM((NOUT, BM, F_HID), jnp.float32),
            pltpu.VMEM((NOUT, BM, F_HID), jnp.float32),
            pltpu.SemaphoreType.DMA((NBUF,)),
            pltpu.SemaphoreType.DMA((2, NOUT)),
        ],
    )(x, W, b2, adj)
    return (logp, embed)


# speedup vs baseline: 1.0175x; 1.0175x over previous
import jax
import jax.numpy as jnp
from jax.experimental import pallas as pl
from jax.experimental.pallas import tpu as pltpu

N = 10000
F_IN = 128
F_HID = 64
BM = 400
B0 = 80
B1 = 320
NLOOP = N // BM - 1


def _gcn_kernel(x_ref, w_ref, b_ref, adj_ref, logp_ref, embed_ref,
                buf_ref, support_ref, x_vmem_ref, sem_ref, xsem_ref):
    def block_copy(row_start, rows, slot):
        return pltpu.make_async_copy(
            adj_ref.at[pl.ds(row_start, rows), :],
            buf_ref.at[slot, pl.ds(0, rows), :],
            sem_ref.at[slot],
        )

    def emit(out_rows, row_start, rows):
        out = out_rows + b_ref[:]
        embed_ref[pl.ds(row_start, rows), :] = out
        m = jnp.max(out, axis=1, keepdims=True)
        lse = jnp.log(jnp.sum(jnp.exp(out - m), axis=1, keepdims=True)) + m
        logp_ref[pl.ds(row_start, rows), :] = out - lse

    x_copy = pltpu.make_async_copy(x_ref, x_vmem_ref, xsem_ref)
    x_copy.start()
    block_copy(0, B0, 0).start()
    block_copy(B0, B1, 1).start()

    x_copy.wait()
    support_ref[:] = jnp.dot(
        x_vmem_ref[:], w_ref[:], preferred_element_type=jnp.float32
    )

    block_copy(0, B0, 0).wait()
    out0 = jnp.dot(buf_ref[0, 0:B0, :], support_ref[:],
                   preferred_element_type=jnp.float32)
    emit(out0, 0, B0)
    block_copy(BM, BM, 0).start()

    block_copy(B0, B1, 1).wait()
    out1 = jnp.dot(buf_ref[1, 0:B1, :], support_ref[:],
                   preferred_element_type=jnp.float32)
    emit(out1, B0, B1)
    block_copy(2 * BM, BM, 1).start()

    def body(i, _):
        slot = jax.lax.rem(i, 2)
        row_start = (i + 1) * BM
        block_copy(row_start, BM, slot).wait()
        out = jnp.dot(buf_ref[slot, :, :], support_ref[:],
                      preferred_element_type=jnp.float32)
        emit(out, row_start, BM)

        @pl.when(i + 2 < NLOOP)
        def _():
            block_copy((i + 3) * BM, BM, slot).start()

        return 0

    jax.lax.fori_loop(0, NLOOP, body, 0)


def kernel(x, adj, W, b):
    b2 = b.reshape(1, F_HID)
    logp, embed = pl.pallas_call(
        _gcn_kernel,
        in_specs=[
            pl.BlockSpec(memory_space=pl.ANY),
            pl.BlockSpec(memory_space=pltpu.VMEM),
            pl.BlockSpec(memory_space=pltpu.VMEM),
            pl.BlockSpec(memory_space=pl.ANY),
        ],
        out_specs=[
            pl.BlockSpec(memory_space=pltpu.VMEM),
            pl.BlockSpec(memory_space=pltpu.VMEM),
        ],
        out_shape=[
            jax.ShapeDtypeStruct((N, F_HID), jnp.float32),
            jax.ShapeDtypeStruct((N, F_HID), jnp.float32),
        ],
        scratch_shapes=[
            pltpu.VMEM((2, BM, N), jnp.float32),
            pltpu.VMEM((N, F_HID), jnp.float32),
            pltpu.VMEM((N, F_IN), jnp.float32),
            pltpu.SemaphoreType.DMA((2,)),
            pltpu.SemaphoreType.DMA,
        ],
    )(x, W, b2, adj)
    return (logp, embed)


# speedup vs baseline: 1.0361x; 1.0183x over previous
import jax
import jax.numpy as jnp
from jax.experimental import pallas as pl
from jax.experimental.pallas import tpu as pltpu

N = 10000
F_IN = 128
F_HID = 64
BM = 400


def _gcn_kernel(x_ref, w_ref, b_ref, adj_ref, logp_ref, embed_ref, support_ref):
    @pl.when(pl.program_id(0) == 0)
    def _():
        support_ref[:] = jnp.dot(
            x_ref[:], w_ref[:], preferred_element_type=jnp.float32
        )

    out = jnp.dot(adj_ref[:], support_ref[:], preferred_element_type=jnp.float32)
    out = out + b_ref[:]
    embed_ref[:] = out
    m = jnp.max(out, axis=1, keepdims=True)
    lse = jnp.log(jnp.sum(jnp.exp(out - m), axis=1, keepdims=True)) + m
    logp_ref[:] = out - lse


def kernel(x, adj, W, b):
    b2 = b.reshape(1, F_HID)
    grid = (N // BM,)
    logp, embed = pl.pallas_call(
        _gcn_kernel,
        grid=grid,
        in_specs=[
            pl.BlockSpec((N, F_IN), lambda i: (0, 0)),
            pl.BlockSpec((F_IN, F_HID), lambda i: (0, 0)),
            pl.BlockSpec((1, F_HID), lambda i: (0, 0)),
            pl.BlockSpec((BM, N), lambda i: (i, 0)),
        ],
        out_specs=[
            pl.BlockSpec((BM, F_HID), lambda i: (i, 0)),
            pl.BlockSpec((BM, F_HID), lambda i: (i, 0)),
        ],
        out_shape=[
            jax.ShapeDtypeStruct((N, F_HID), jnp.float32),
            jax.ShapeDtypeStruct((N, F_HID), jnp.float32),
        ],
        scratch_shapes=[pltpu.VMEM((N, F_HID), jnp.float32)],
        compiler_params=pltpu.CompilerParams(
            dimension_semantics=("arbitrary",),
        ),
    )(x, W, b2, adj)
    return (logp, embed)
